# Initial kernel scaffold; baseline (speedup 1.0000x reference)
import jax
import jax.numpy as jnp
from jax.experimental import pallas as pl


def kernel(input_ids, params):
    raise NotImplementedError("write your pallas kernel here")


# speedup vs baseline: 1.8546x; 1.8546x over previous
import jax
import jax.numpy as jnp
from jax import lax
from jax.experimental import pallas as pl
from jax.experimental.pallas import tpu as pltpu

_TEMP = 0.7
_ENTROPY_TRIG = 2.2
_EPS = 1e-5
_E = 16
_VT = 3200


def _ln(x, w, b):
    mu = jnp.mean(x, axis=-1, keepdims=True)
    var = jnp.mean((x - mu) ** 2, axis=-1, keepdims=True)
    return (x - mu) / jnp.sqrt(var + _EPS) * w + b


_EMB_TILE = 256


def _embed_body(ids_ref, tok_hbm, pos_ref, out_ref, sem):
    i = pl.program_id(0)
    base = i * _EMB_TILE
    copies = []
    for mi in range(_EMB_TILE):
        c = pltpu.make_async_copy(
            tok_hbm.at[ids_ref[base + mi]], out_ref.at[mi], sem)
        c.start()
        copies.append(c)
    for c in copies:
        c.wait()
    out_ref[...] = out_ref[...] + pos_ref[...]


def _embed(ids, tok, pos):
    n = ids.shape[0]
    t = pos.shape[0]
    d = tok.shape[1]
    tok3 = tok.reshape(tok.shape[0], 1, d)
    pos3 = pos.reshape(t, 1, d)
    n_tiles = n // _EMB_TILE
    pos_tiles = t // _EMB_TILE
    out = pl.pallas_call(
        _embed_body,
        out_shape=jax.ShapeDtypeStruct((n, 1, d), jnp.float32),
        grid_spec=pltpu.PrefetchScalarGridSpec(
            num_scalar_prefetch=1,
            grid=(n_tiles,),
            in_specs=[
                pl.BlockSpec(memory_space=pl.ANY),
                pl.BlockSpec((_EMB_TILE, 1, d),
                             lambda i, ids_r: (i % pos_tiles, 0, 0)),
            ],
            out_specs=pl.BlockSpec((_EMB_TILE, 1, d),
                                   lambda i, ids_r: (i, 0, 0)),
            scratch_shapes=[pltpu.SemaphoreType.DMA],
        ),
        compiler_params=pltpu.CompilerParams(
            dimension_semantics=("arbitrary",)),
        name="embed_gather",
    )(ids, tok3, pos3)
    return out.reshape(n, d)


def _moe_body(x_ref, lnw_ref, lnb_ref, rw_ref, rb_ref,
              w1_ref, b1_ref, w2_ref, b2_ref,
              moe_ref, hn_ref, wf_ref):
    e = pl.program_id(1)

    @pl.when(e == 0)
    def _():
        x = x_ref[...]
        hn = _ln(x, lnw_ref[...], lnb_ref[...])
        hn_ref[...] = hn
        logits = jnp.dot(hn, rw_ref[...],
                         preferred_element_type=jnp.float32) + rb_ref[...]
        g = jax.nn.softmax(logits / _TEMP, axis=-1)
        iota = lax.broadcasted_iota(jnp.int32, g.shape, 1)
        m1 = jnp.max(g, axis=-1, keepdims=True)
        a1 = jnp.min(jnp.where(g == m1, iota, _E), axis=-1, keepdims=True)
        g2 = jnp.where(iota == a1, -jnp.inf, g)
        m2 = jnp.max(g2, axis=-1, keepdims=True)
        a2 = jnp.min(jnp.where(g2 == m2, iota, _E), axis=-1, keepdims=True)
        wf_ref[...] = (jnp.where(iota == a1, m1, 0.0)
                       + jnp.where(iota == a2, m2, 0.0))
        moe_ref[...] = jnp.zeros_like(moe_ref)

    hn = hn_ref[...]
    mid = jax.nn.silu(
        jnp.dot(hn, w1_ref[0], preferred_element_type=jnp.float32)
        + b1_ref[0])
    eo = (jnp.dot(mid, w2_ref[0], preferred_element_type=jnp.float32)
          + b2_ref[0])
    iota = lax.broadcasted_iota(jnp.int32, wf_ref.shape, 1)
    wcol = jnp.sum(wf_ref[...] * (iota == e).astype(jnp.float32),
                   axis=-1, keepdims=True)
    moe_ref[...] = moe_ref[...] + wcol * eo


def _moe(x, lp):
    n, d = x.shape
    h4 = lp["e_w1"].shape[2]
    tt = n // 2
    return pl.pallas_call(
        _moe_body,
        out_shape=jax.ShapeDtypeStruct((n, d), jnp.float32),
        grid=(2, _E),
        in_specs=[
            pl.BlockSpec((tt, d), lambda t, e: (t, 0)),
            pl.BlockSpec((1, d), lambda t, e: (0, 0)),
            pl.BlockSpec((1, d), lambda t, e: (0, 0)),
            pl.BlockSpec((d, _E), lambda t, e: (0, 0)),
            pl.BlockSpec((1, _E), lambda t, e: (0, 0)),
            pl.BlockSpec((1, d, h4), lambda t, e: (e, 0, 0)),
            pl.BlockSpec((1, 1, h4), lambda t, e: (e, 0, 0)),
            pl.BlockSpec((1, h4, d), lambda t, e: (e, 0, 0)),
            pl.BlockSpec((1, 1, d), lambda t, e: (e, 0, 0)),
        ],
        out_specs=pl.BlockSpec((tt, d), lambda t, e: (t, 0)),
        scratch_shapes=[
            pltpu.VMEM((tt, d), jnp.float32),
            pltpu.VMEM((tt, _E), jnp.float32),
        ],
        compiler_params=pltpu.CompilerParams(
            dimension_semantics=("parallel", "arbitrary"),
            vmem_limit_bytes=50 * 1024 * 1024,
        ),
        name="moe_experts",
    )(x, lp["ln_in_w"].reshape(1, d), lp["ln_in_b"].reshape(1, d),
      lp["router_w"], lp["router_b"].reshape(1, _E),
      lp["e_w1"], lp["e_b1"].reshape(_E, 1, h4),
      lp["e_w2"], lp["e_b2"].reshape(_E, 1, d))


def _ffn_body(x_ref, moe_ref, lnw_ref, lnb_ref,
              sw1_ref, sb1_ref, sw2_ref, sb2_ref,
              flnw_ref, flnb_ref, fw1_ref, fb1_ref, fw2_ref, fb2_ref,
              out_ref):
    x = x_ref[...]
    hn = _ln(x, lnw_ref[...], lnb_ref[...])
    shared = (jnp.dot(
        jax.nn.silu(jnp.dot(hn, sw1_ref[...],
                            preferred_element_type=jnp.float32)
                    + sb1_ref[...]),
        sw2_ref[...], preferred_element_type=jnp.float32)
        + sb2_ref[...]) * 0.25
    moe = moe_ref[...] + shared
    fh = _ln(moe, flnw_ref[...], flnb_ref[...])
    ff = (jnp.dot(
        jax.nn.silu(jnp.dot(fh, fw1_ref[...],
                            preferred_element_type=jnp.float32)
                    + fb1_ref[...]),
        fw2_ref[...], preferred_element_type=jnp.float32)
        + fb2_ref[...])
    out_ref[...] = x + moe + ff


def _ffn(x, moe, lp):
    n, d = x.shape
    d2 = lp["s_w1"].shape[1]
    h4 = lp["ff_w1"].shape[1]
    tt = n // 4
    return pl.pallas_call(
        _ffn_body,
        out_shape=jax.ShapeDtypeStruct((n, d), jnp.float32),
        grid=(4,),
        in_specs=[
            pl.BlockSpec((tt, d), lambda t: (t, 0)),
            pl.BlockSpec((tt, d), lambda t: (t, 0)),
            pl.BlockSpec((1, d), lambda t: (0, 0)),
            pl.BlockSpec((1, d), lambda t: (0, 0)),
            pl.BlockSpec((d, d2), lambda t: (0, 0)),
            pl.BlockSpec((1, d2), lambda t: (0, 0)),
            pl.BlockSpec((d2, d), lambda t: (0, 0)),
            pl.BlockSpec((1, d), lambda t: (0, 0)),
            pl.BlockSpec((1, d), lambda t: (0, 0)),
            pl.BlockSpec((1, d), lambda t: (0, 0)),
            pl.BlockSpec((d, h4), lambda t: (0, 0)),
            pl.BlockSpec((1, h4), lambda t: (0, 0)),
            pl.BlockSpec((h4, d), lambda t: (0, 0)),
            pl.BlockSpec((1, d), lambda t: (0, 0)),
        ],
        out_specs=pl.BlockSpec((tt, d), lambda t: (t, 0)),
        compiler_params=pltpu.CompilerParams(
            dimension_semantics=("parallel",),
            vmem_limit_bytes=50 * 1024 * 1024,
        ),
        name="shared_ffn",
    )(x, moe, lp["ln_in_w"].reshape(1, d), lp["ln_in_b"].reshape(1, d),
      lp["s_w1"], lp["s_b1"].reshape(1, d2),
      lp["s_w2"], lp["s_b2"].reshape(1, d),
      lp["ff_ln_w"].reshape(1, d), lp["ff_ln_b"].reshape(1, d),
      lp["ff_w1"], lp["ff_b1"].reshape(1, h4),
      lp["ff_w2"], lp["ff_b2"].reshape(1, d))


def _final_ln_body(x_ref, w_ref, b_ref, out_ref):
    out_ref[...] = _ln(x_ref[...], w_ref[...], b_ref[...])


def _final_ln(x, w, b):
    n, d = x.shape
    tt = n // 4
    return pl.pallas_call(
        _final_ln_body,
        out_shape=jax.ShapeDtypeStruct((n, d), jnp.float32),
        grid=(4,),
        in_specs=[
            pl.BlockSpec((tt, d), lambda t: (t, 0)),
            pl.BlockSpec((1, d), lambda t: (0, 0)),
            pl.BlockSpec((1, d), lambda t: (0, 0)),
        ],
        out_specs=pl.BlockSpec((tt, d), lambda t: (t, 0)),
        compiler_params=pltpu.CompilerParams(
            dimension_semantics=("parallel",)),
        name="final_ln",
    )(x, w.reshape(1, d), b.reshape(1, d))


def _head_body(h_ref, w_ref, out_ref):
    out_ref[...] = jnp.dot(h_ref[...], w_ref[...],
                           preferred_element_type=jnp.float32)


def _head(h, w):
    n, d = h.shape
    v = w.shape[1]
    tt = 256
    return pl.pallas_call(
        _head_body,
        out_shape=jax.ShapeDtypeStruct((n, v), jnp.float32),
        grid=(v // _VT, n // tt),
        in_specs=[
            pl.BlockSpec((tt, d), lambda vi, t: (t, 0)),
            pl.BlockSpec((d, _VT), lambda vi, t: (0, vi)),
        ],
        out_specs=pl.BlockSpec((tt, _VT), lambda vi, t: (t, vi)),
        compiler_params=pltpu.CompilerParams(
            dimension_semantics=("parallel", "arbitrary")),
        name="lm_head",
    )(h, w)


def _head_noise_body(h_ref, nz_ref, w_ref, out_ref):
    out_ref[...] = jnp.dot(h_ref[...] + nz_ref[...], w_ref[...],
                           preferred_element_type=jnp.float32)


def _head_noise(h, nz, w):
    n, d = h.shape
    v = w.shape[1]
    tt = 256
    return pl.pallas_call(
        _head_noise_body,
        out_shape=jax.ShapeDtypeStruct((n, v), jnp.float32),
        grid=(v // _VT, n // tt),
        in_specs=[
            pl.BlockSpec((tt, d), lambda vi, t: (t, 0)),
            pl.BlockSpec((tt, d), lambda vi, t: (t, 0)),
            pl.BlockSpec((d, _VT), lambda vi, t: (0, vi)),
        ],
        out_specs=pl.BlockSpec((tt, _VT), lambda vi, t: (t, vi)),
        compiler_params=pltpu.CompilerParams(
            dimension_semantics=("parallel", "arbitrary")),
        name="lm_head_noise",
    )(h, nz, w)


def _hlast_body(hl_ref, w_ref, out_ref, acc_ref):
    vi = pl.program_id(0)
    nv = pl.num_programs(0)
    acc_ref[vi] = jnp.dot(hl_ref[...], w_ref[...],
                          preferred_element_type=jnp.float32)

    @pl.when(vi == nv - 1)
    def _():
        full = acc_ref[...]
        m = jnp.max(jnp.max(full, axis=0), axis=-1, keepdims=True)
        p = jnp.exp(full - m[None])
        z = jnp.sum(jnp.sum(p, axis=0), axis=-1, keepdims=True)
        pn = p / z[None]
        term = pn * jnp.log(jnp.maximum(pn, 1e-9))
        ht = -jnp.sum(jnp.sum(term, axis=0), axis=-1)
        out_ref[...] = jnp.mean(ht)[None, None]


def _hlast_entropy(hl, w):
    rows, d = hl.shape
    v = w.shape[1]
    nv = v // _VT
    return pl.pallas_call(
        _hlast_body,
        out_shape=jax.ShapeDtypeStruct((1, 1), jnp.float32),
        grid=(nv,),
        in_specs=[
            pl.BlockSpec((rows, d), lambda vi: (0, 0)),
            pl.BlockSpec((d, _VT), lambda vi: (0, vi)),
        ],
        out_specs=pl.BlockSpec((1, 1), lambda vi: (0, 0)),
        scratch_shapes=[pltpu.VMEM((nv, rows, _VT), jnp.float32)],
        name="last_token_entropy",
    )(hl, w)


def _vote_stats_body(h_ref, n0_ref, n1_ref, w_ref, out_ref):
    hb = h_ref[...]
    wb = w_ref[...]
    for i, nz in enumerate((n0_ref, n1_ref)):
        c = jnp.dot(hb + nz[...], wb, preferred_element_type=jnp.float32)
        m = jnp.max(c, axis=-1, keepdims=True)
        ez = jnp.exp(c - m)
        z = jnp.sum(ez, axis=-1, keepdims=True)
        sx = jnp.sum(c * ez, axis=-1, keepdims=True)
        out_ref[0, :, 3 * i + 0:3 * i + 1] = m
        out_ref[0, :, 3 * i + 1:3 * i + 2] = z
        out_ref[0, :, 3 * i + 2:3 * i + 3] = sx


def _vote_stats(h, n0, n1, w):
    n, d = h.shape
    v = w.shape[1]
    nv = v // _VT
    tt = 256
    nt = n // tt
    return pl.pallas_call(
        _vote_stats_body,
        out_shape=jax.ShapeDtypeStruct((nv, n, 6), jnp.float32),
        grid=(nv, nt),
        in_specs=[
            pl.BlockSpec((tt, d), lambda vi, t: (t, 0)),
            pl.BlockSpec((tt, d), lambda vi, t: (t, 0)),
            pl.BlockSpec((tt, d), lambda vi, t: (t, 0)),
            pl.BlockSpec((d, _VT), lambda vi, t: (0, vi)),
        ],
        out_specs=pl.BlockSpec((1, tt, 6), lambda vi, t: (vi, t, 0)),
        compiler_params=pltpu.CompilerParams(
            dimension_semantics=("parallel", "arbitrary")),
        name="vote_entropy_stats",
    )(h, n0, n1, w)


def _vote_merge_body(st_ref, out_ref):
    st = st_ref[...]
    res = []
    for i in range(2):
        m_v = st[:, :, 3 * i + 0:3 * i + 1]
        z_v = st[:, :, 3 * i + 1:3 * i + 2]
        sx_v = st[:, :, 3 * i + 2:3 * i + 3]
        m = jnp.max(m_v, axis=0)
        scale = jnp.exp(m_v - m[None])
        zz = jnp.sum(z_v * scale, axis=0)
        sxx = jnp.sum(sx_v * scale, axis=0)
        ht = m + jnp.log(zz) - sxx / zz
        res.append(jnp.mean(ht).reshape(1, 1))
    out_ref[...] = jnp.concatenate(res, axis=1)


def _vote_merge(st):
    return pl.pallas_call(
        _vote_merge_body,
        out_shape=jax.ShapeDtypeStruct((1, 2), jnp.float32),
        name="vote_entropy_merge",
    )(st)


def kernel(input_ids, params):
    b, t = input_ids.shape
    tok = params["tok"]
    d = tok.shape[1]
    n = b * t
    ids = input_ids.reshape(n).astype(jnp.int32)

    x = _embed(ids, tok, params["pos"][:t])
    for lp in params["layers"]:
        moe = _moe(x, lp)
        x = _ffn(x, moe, lp)
    h = _final_ln(x, params["norm_w"], params["norm_b"])

    w = params["lm_head"]
    v = w.shape[1]
    h_bt = h.reshape(b, t, d)
    h_last = h_bt[:, -1, :]
    trig = _hlast_entropy(h_last, w)[0, 0]

    def _novote(_):
        return _head(h, w)

    def _vote(_):
        nk = jax.random.split(jax.random.key(1234), 2)
        n0 = 0.01 * jax.random.normal(nk[0], h_bt.shape, h_bt.dtype)
        n1 = 0.01 * jax.random.normal(nk[1], h_bt.shape, h_bt.dtype)
        n0 = n0.reshape(n, d)
        n1 = n1.reshape(n, d)
        st = _vote_stats(h, n0, n1, w)
        s = _vote_merge(st)
        nz = jnp.where(s[0, 0] <= s[0, 1], n0, n1)
        return _head_noise(h, nz, w)

    logits = lax.cond(trig >= _ENTROPY_TRIG, _vote, _novote, None)
    return logits.reshape(b, t, v)
